# Initial kernel scaffold; baseline (speedup 1.0000x reference)
import jax
import jax.numpy as jnp
from jax.experimental import pallas as pl


def kernel(table, flat_charges, center_idx):
    raise NotImplementedError("write your pallas kernel here")


# speedup vs baseline: 1.8923x; 1.8923x over previous
import functools

import jax
import jax.numpy as jnp
from jax import lax
from jax.experimental import pallas as pl
from jax.experimental.pallas import tpu as pltpu
from jax.experimental.pallas import tpu_sc as plsc

N_NUC = 10000
N_CENTER = 320000
MAX_CHARGE = 100
FEAT = 128

NC, NS = 2, 16
NW = NC * NS
PER_W = N_CENTER // NW
CHUNK = 400
SUB = 80
NSUB = CHUNK // SUB
NCHUNK = PER_W // CHUNK

_mesh = plsc.VectorSubcoreMesh(core_axis_name="c", subcore_axis_name="s")


@functools.partial(
    pl.kernel,
    out_type=jax.ShapeDtypeStruct((N_CENTER, FEAT), jnp.float32),
    mesh=_mesh,
    scratch_types=[
        pltpu.VMEM((CHUNK,), jnp.int32),
        pltpu.VMEM((CHUNK,), jnp.int32),
        pltpu.VMEM((CHUNK, FEAT), jnp.float32),
        pltpu.SemaphoreType.DMA,
    ],
)
def _two_level_gather(table_hbm, charges_hbm, cidx_hbm, out_hbm,
                      cidx_v, chg_v, rows_v, sem):
    wid = lax.axis_index("s") * NC + lax.axis_index("c")
    base = wid * PER_W

    def step(ci, _):
        off = base + ci * CHUNK
        pltpu.sync_copy(cidx_hbm.at[pl.ds(off, CHUNK)], cidx_v)
        for j in range(NSUB):
            sl = pl.ds(j * SUB, SUB)
            pltpu.async_copy(charges_hbm.at[cidx_v.at[sl]], chg_v.at[sl], sem).wait()
            pltpu.async_copy(table_hbm.at[chg_v.at[sl]], rows_v.at[sl], sem).wait()
        pltpu.sync_copy(rows_v, out_hbm.at[pl.ds(off, CHUNK)])
        return ()

    lax.fori_loop(0, NCHUNK, step, ())


def kernel(table, flat_charges, center_idx):
    return _two_level_gather(
        table,
        flat_charges.astype(jnp.int32),
        center_idx.astype(jnp.int32),
    )


# speedup vs baseline: 2.1995x; 1.1624x over previous
import functools

import jax
import jax.numpy as jnp
from jax import lax
from jax.experimental import pallas as pl
from jax.experimental.pallas import tpu as pltpu
from jax.experimental.pallas import tpu_sc as plsc

N_NUC = 10000
N_CENTER = 320000
MAX_CHARGE = 100
FEAT = 128

NC, NS = 2, 16
NW = NC * NS
PER_W = N_CENTER // NW
CHUNK = 400
NCHUNK = PER_W // CHUNK
NB = 2
NRING = NCHUNK // NB

_mesh = plsc.VectorSubcoreMesh(core_axis_name="c", subcore_axis_name="s")


@functools.partial(
    pl.kernel,
    out_type=jax.ShapeDtypeStruct((N_CENTER, FEAT), jnp.float32),
    mesh=_mesh,
    scratch_types=[
        pltpu.VMEM_SHARED((N_NUC,), jnp.int32),
        pltpu.VMEM((CHUNK,), jnp.int32),
        pltpu.VMEM((CHUNK,), jnp.int32),
        pltpu.VMEM((CHUNK,), jnp.int32),
        pltpu.VMEM((CHUNK,), jnp.int32),
        pltpu.VMEM((CHUNK, FEAT), jnp.float32),
        pltpu.VMEM((CHUNK, FEAT), jnp.float32),
        pltpu.SemaphoreType.DMA,
        pltpu.SemaphoreType.DMA,
        pltpu.SemaphoreType.DMA,
        pltpu.SemaphoreType.DMA,
        pltpu.SemaphoreType.DMA,
    ],
)
def _two_level_gather(table_hbm, charges_hbm, cidx_hbm, out_hbm,
                      fc_s, cidx0, cidx1, chg0, chg1, rows0, rows1,
                      sem_i0, sem_i1, sem_g, sem_s0, sem_s1):
    wid = lax.axis_index("s") * NC + lax.axis_index("c")
    base = wid * PER_W
    cidx_v = (cidx0, cidx1)
    chg_v = (chg0, chg1)
    rows_v = (rows0, rows1)
    sem_i = (sem_i0, sem_i1)
    sem_s = (sem_s0, sem_s1)

    @pl.when(lax.axis_index("s") == 0)
    def _():
        pltpu.sync_copy(charges_hbm, fc_s)
    plsc.subcore_barrier()

    def wait_cidx(b):
        pltpu.make_async_copy(
            cidx_hbm.at[pl.ds(0, CHUNK)], cidx_v[b], sem_i[b]).wait()

    def wait_store(b):
        pltpu.make_async_copy(
            rows_v[b], out_hbm.at[pl.ds(0, CHUNK)], sem_s[b]).wait()

    def do_chunk(c, b, prefetch_next):
        off = base + c * CHUNK
        wait_cidx(b)
        pltpu.async_copy(fc_s.at[cidx_v[b]], chg_v[b], sem_g).wait()
        gat = pltpu.async_copy(table_hbm.at[chg_v[b]], rows_v[b], sem_g)
        if prefetch_next:
            nb = 1 - b
            pltpu.async_copy(
                cidx_hbm.at[pl.ds(off + CHUNK, CHUNK)], cidx_v[nb], sem_i[nb])
        gat.wait()
        pltpu.async_copy(rows_v[b], out_hbm.at[pl.ds(off, CHUNK)], sem_s[b])

    pltpu.async_copy(cidx_hbm.at[pl.ds(base, CHUNK)], cidx_v[0], sem_i[0])

    @pl.loop(0, NRING)
    def _(g):
        for b in range(NB):
            @pl.when(g > 0)
            def _():
                wait_store(b)
            do_chunk(g * NB + b, b, prefetch_next=True)

    wait_store(0)
    do_chunk(NCHUNK - 1, 0, prefetch_next=False)
    wait_store(0)
    wait_store(1)


def kernel(table, flat_charges, center_idx):
    return _two_level_gather(
        table,
        flat_charges.astype(jnp.int32),
        center_idx.astype(jnp.int32),
    )
